# baseline (speedup 1.0000x reference)
import functools

import jax
import jax.numpy as jnp
from jax import lax
from jax.experimental import pallas as pl
from jax.experimental.pallas import tpu as pltpu
from jax.experimental.pallas import tpu_sc as plsc

_NUM_CORES = 2
_NUM_SUBCORES = 16
_NUM_WORKERS = _NUM_CORES * _NUM_SUBCORES
_CHUNK = 16384
_NBUF = 6


def kernel(inputs):
    B, R, C = inputs.shape
    row_words = R * C
    n = B * row_words
    workers_per_row = _NUM_WORKERS // B
    wpw = row_words // workers_per_row
    nchunks = wpw // _CHUNK
    assert wpw % _CHUNK == 0

    perm = jax.random.permutation(jax.random.key(42), B)
    wid = (
        jnp.arange(_NUM_SUBCORES, dtype=jnp.int32)[None, :] * _NUM_CORES
        + jnp.arange(_NUM_CORES, dtype=jnp.int32)[:, None]
    )
    src_off = (
        perm.astype(jnp.int32)[wid // workers_per_row] * row_words
        + (wid % workers_per_row) * wpw
    ).astype(jnp.int32)
    src_off = jnp.broadcast_to(src_off[:, :, None], (_NUM_CORES, _NUM_SUBCORES, 16))
    src_off = src_off.astype(jnp.int32)

    flat = inputs.reshape(n)
    mesh = plsc.VectorSubcoreMesh(core_axis_name="c", subcore_axis_name="s")

    @functools.partial(
        pl.kernel,
        out_type=jax.ShapeDtypeStruct((n,), jnp.float32),
        mesh=mesh,
        scratch_types=[
            pltpu.VMEM((16,), jnp.int32),
            *[pltpu.VMEM((_CHUNK,), jnp.float32) for _ in range(_NBUF)],
            *[pltpu.SemaphoreType.DMA for _ in range(2 * _NBUF)],
        ],
    )
    def run(in_hbm, off_hbm, out_hbm, off_v, *bufs_and_sems):
        bufs = bufs_and_sems[:_NBUF]
        lsems = bufs_and_sems[_NBUF : 2 * _NBUF]
        ssems = bufs_and_sems[2 * _NBUF :]
        cid = lax.axis_index("c")
        sid = lax.axis_index("s")
        w = sid * _NUM_CORES + cid
        pltpu.sync_copy(off_hbm.at[cid, sid], off_v)
        src = off_v[...][0]
        dst = w * wpw

        def src_at(k):
            return in_hbm.at[pl.ds(pl.multiple_of(src + k * _CHUNK, 8), _CHUNK)]

        def dst_at(k):
            return out_hbm.at[pl.ds(pl.multiple_of(dst + k * _CHUNK, 8), _CHUNK)]

        loads = [
            pltpu.async_copy(src_at(k), bufs[k], lsems[k])
            for k in range(min(_NBUF, nchunks))
        ]
        stores = []
        for k in range(nchunks):
            b = k % _NBUF
            loads[k].wait()
            stores.append(pltpu.async_copy(bufs[b], dst_at(k), ssems[b]))
            nk = k + _NBUF
            if nk < nchunks:
                stores[k].wait()
                loads.append(pltpu.async_copy(src_at(nk), bufs[b], lsems[b]))
        for k in range(max(0, nchunks - _NBUF), nchunks):
            stores[k].wait()

    return run(flat, src_off).reshape(B, R, C)


# speedup vs baseline: 2.9610x; 2.9610x over previous
import functools

import jax
import jax.numpy as jnp
from jax import lax
from jax.experimental import pallas as pl
from jax.experimental.pallas import tpu as pltpu
from jax.experimental.pallas import tpu_sc as plsc

_NUM_CORES = 2
_NUM_SUBCORES = 16
_NUM_WORKERS = _NUM_CORES * _NUM_SUBCORES
_CHUNK_ROWS = 8
_NBUF = 3


def kernel(inputs):
    B, R, C = inputs.shape
    workers_per_row = _NUM_WORKERS // B
    rpw = R // workers_per_row
    nchunks = rpw // _CHUNK_ROWS
    assert rpw % _CHUNK_ROWS == 0

    perm = jax.random.permutation(jax.random.key(42), B)
    wid = (
        jnp.arange(_NUM_SUBCORES, dtype=jnp.int32)[None, :] * _NUM_CORES
        + jnp.arange(_NUM_CORES, dtype=jnp.int32)[:, None]
    )
    src_batch = perm.astype(jnp.int32)[wid // workers_per_row]
    src_batch = jnp.broadcast_to(
        src_batch[:, :, None], (_NUM_CORES, _NUM_SUBCORES, 16)
    ).astype(jnp.int32)

    mesh = plsc.VectorSubcoreMesh(core_axis_name="c", subcore_axis_name="s")

    @functools.partial(
        pl.kernel,
        out_type=jax.ShapeDtypeStruct((B, R, C), jnp.float32),
        mesh=mesh,
        scratch_types=[
            pltpu.VMEM((16,), jnp.int32),
            *[pltpu.VMEM((_CHUNK_ROWS, C), jnp.float32) for _ in range(_NBUF)],
            *[pltpu.SemaphoreType.DMA for _ in range(2 * _NBUF)],
        ],
    )
    def run(in_hbm, src_hbm, out_hbm, idx_v, *bufs_and_sems):
        bufs = bufs_and_sems[:_NBUF]
        lsems = bufs_and_sems[_NBUF : 2 * _NBUF]
        ssems = bufs_and_sems[2 * _NBUF :]
        cid = lax.axis_index("c")
        sid = lax.axis_index("s")
        w = sid * _NUM_CORES + cid
        pltpu.sync_copy(src_hbm.at[cid, sid], idx_v)
        src_b = idx_v[...][0]
        dst_b = w // workers_per_row
        r0 = (w % workers_per_row) * rpw

        def src_at(k):
            return in_hbm.at[
                src_b, pl.ds(pl.multiple_of(r0 + k * _CHUNK_ROWS, 8), _CHUNK_ROWS), :
            ]

        def dst_at(k):
            return out_hbm.at[
                dst_b, pl.ds(pl.multiple_of(r0 + k * _CHUNK_ROWS, 8), _CHUNK_ROWS), :
            ]

        loads = [
            pltpu.async_copy(src_at(k), bufs[k], lsems[k])
            for k in range(min(_NBUF, nchunks))
        ]
        stores = []
        for k in range(nchunks):
            b = k % _NBUF
            loads[k].wait()
            stores.append(pltpu.async_copy(bufs[b], dst_at(k), ssems[b]))
            nk = k + _NBUF
            if nk < nchunks:
                stores[k].wait()
                loads.append(pltpu.async_copy(src_at(nk), bufs[b], lsems[b]))
        for k in range(max(0, nchunks - _NBUF), nchunks):
            stores[k].wait()

    return run(inputs, src_batch)
